# baseline (speedup 1.0000x reference)
import jax
import jax.numpy as jnp
from jax.experimental import pallas as pl
from jax.experimental.pallas import tpu as pltpu
from jax.experimental.pallas import tpu_sc as plsc

VOCAB = 128
DIM = 32
WINDOW = 1024


def kernel(x, table):
    orig_shape = x.shape
    n = x.size
    idx = x.reshape(1, n).astype(jnp.int32)
    table = table.astype(jnp.float32)

    mesh = plsc.VectorSubcoreMesh(core_axis_name="core",
                                  subcore_axis_name="subcore")

    @pl.kernel(out_type=jax.ShapeDtypeStruct((n, DIM), jnp.float32),
               mesh=mesh,
               compiler_params=pltpu.CompilerParams(use_tc_tiling_on_sc=False))
    def gather_kernel(table_hbm, i_hbm, o_hbm):
        def body(i_vmem, o_vmem):
            pltpu.sync_copy(table_hbm.at[i_vmem.at[0]], o_vmem)

        pltpu.emit_pipeline(
            body,
            grid=(n // WINDOW,),
            in_specs=[pl.BlockSpec((1, WINDOW), lambda i: (0, i))],
            out_specs=[pl.BlockSpec((WINDOW, DIM), lambda i: (i, 0))],
            core_axis_name=("core", "subcore"),
            dimension_semantics=(pltpu.PARALLEL,),
        )(i_hbm, o_hbm)

    out = gather_kernel(table, idx)
    return out.reshape(*orig_shape, DIM)


# speedup vs baseline: 1.2882x; 1.2882x over previous
import jax
import jax.numpy as jnp
from jax import lax
from jax.experimental import pallas as pl
from jax.experimental.pallas import tpu as pltpu
from jax.experimental.pallas import tpu_sc as plsc

VOCAB = 128
DIM = 32
WINDOW = 1024
UNROLL = 16


def kernel(x, table):
    orig_shape = x.shape
    n = x.size
    idx = x.reshape(1, n).astype(jnp.int32)
    tab_flat = table.astype(jnp.float32).reshape(VOCAB * DIM)

    mesh = plsc.VectorSubcoreMesh(core_axis_name="core",
                                  subcore_axis_name="subcore")

    @pl.kernel(out_type=jax.ShapeDtypeStruct((n, DIM), jnp.float32),
               mesh=mesh,
               compiler_params=pltpu.CompilerParams(
                   use_tc_tiling_on_sc=False, needs_layout_passes=False),
               scratch_types=[pltpu.VMEM((VOCAB * DIM,), jnp.float32)])
    def gather_kernel(table_hbm, i_hbm, o_hbm, tab_v):
        pltpu.sync_copy(table_hbm, tab_v)
        lanes = lax.iota(jnp.int32, 16)

        def body(i_vmem, o_vmem):
            @pl.loop(0, WINDOW, step=UNROLL)
            def _(i0):
                vbase = i_vmem[0, pl.ds(i0, UNROLL)] * DIM
                for u in range(UNROLL):
                    a0 = vbase[u] + lanes
                    o_vmem[i0 + u, pl.ds(0, 16)] = plsc.load_gather(
                        tab_v, [a0])
                    o_vmem[i0 + u, pl.ds(16, 16)] = plsc.load_gather(
                        tab_v, [a0 + 16])

        pltpu.emit_pipeline(
            body,
            grid=(n // WINDOW,),
            in_specs=[pl.BlockSpec((1, WINDOW), lambda i: (0, i))],
            out_specs=[pl.BlockSpec((WINDOW, DIM), lambda i: (i, 0))],
            core_axis_name=("core", "subcore"),
            dimension_semantics=(pltpu.PARALLEL,),
        )(i_hbm, o_hbm)

    out = gather_kernel(tab_flat, idx)
    return out.reshape(*orig_shape, DIM)
